# baseline (speedup 1.0000x reference)
import functools

import jax
import jax.numpy as jnp
import numpy as np
from jax import lax
from jax.experimental import pallas as pl
from jax.experimental.pallas import tpu as pltpu
from jax.experimental.pallas import tpu_sc as plsc

_BUDGET = 1024
_S = 4096
_D = 4096
_GRID = 16


def _tf2x32(k0, k1, x0, x1):
    c = np.uint32(0x1BD11BDA)
    ks2 = k0 ^ k1 ^ c
    x0 = x0 + k0
    x1 = x1 + k1
    rotations = ((13, 15, 26, 6), (17, 29, 16, 24))
    keys = ((k1, ks2), (ks2, k0), (k0, k1), (k1, ks2), (ks2, k0))
    for i in range(5):
        for r in rotations[i % 2]:
            x0 = x0 + x1
            x1 = (x1 << np.uint32(r)) | (x1 >> np.uint32(32 - r))
            x1 = x1 ^ x0
        ka, kb = keys[i]
        x0 = x0 + ka
        x1 = x1 + kb + np.uint32(i + 1)
    return x0, x1


def _tc_select_body(hash_ref, k_ref, sel_ref, pos_ref, scal_ref, valid_scr):
    g = pl.program_id(0)
    blk = k_ref[...]
    valid_scr[g] = jnp.any(blk != 0.0, axis=2).astype(jnp.int32)

    @pl.when(g == pl.num_programs(0) - 1)
    def _selection():
        valid = valid_scr[...].reshape(32, 128)
        actual_len = jnp.sum(valid)
        compress = (actual_len > _BUDGET).astype(jnp.int32)

        h_u = hash_ref[0].astype(jnp.uint32)
        fk0, fk1 = _tf2x32(jnp.uint32(0), jnp.uint32(0), jnp.uint32(0), h_u)
        idx2d = (lax.broadcasted_iota(jnp.int32, (32, 128), 0) * 128
                 + lax.broadcasted_iota(jnp.int32, (32, 128), 1))
        b1, b2 = _tf2x32(fk0, fk1,
                         jnp.zeros((32, 128), jnp.uint32),
                         idx2d.astype(jnp.uint32))
        bits = b1 ^ b2
        fbits = (bits >> np.uint32(9)) | np.uint32(0x3F800000)
        noise = lax.bitcast_convert_type(fbits, jnp.float32) - 1.0
        noise = jnp.where(valid == 0, -jnp.inf, noise)
        noise = jnp.where(idx2d == 0, jnp.inf, noise)

        def bs_step(_, lohi):
            lo, hi = lohi
            mid = (lo + hi) // 2
            t = lax.bitcast_convert_type(
                jnp.full((32, 128), mid, jnp.int32), jnp.float32)
            cnt = jnp.sum((noise > t).astype(jnp.int32))
            pred = cnt < _BUDGET
            return (jnp.where(pred, lo, mid + 1), jnp.where(pred, mid, hi))

        lo, _hi = lax.fori_loop(0, 30, bs_step,
                                (jnp.int32(0), jnp.int32(0x3F800000)))
        tvec = lax.bitcast_convert_type(
            jnp.full((32, 128), lo, jnp.int32), jnp.float32)
        gt = noise > tvec
        eq = noise == tvec
        n_gt = jnp.sum(gt.astype(jnp.int32))
        r = (_BUDGET - n_gt).astype(jnp.float32)

        up128 = (lax.broadcasted_iota(jnp.int32, (128, 128), 0)
                 < lax.broadcasted_iota(jnp.int32, (128, 128), 1)
                 ).astype(jnp.float32)
        lo32 = (lax.broadcasted_iota(jnp.int32, (32, 32), 1)
                < lax.broadcasted_iota(jnp.int32, (32, 32), 0)
                ).astype(jnp.float32)

        def excl_prefix(mask_f32):
            within = jax.lax.dot_general(
                mask_f32, up128, (((1,), (0,)), ((), ())),
                preferred_element_type=jnp.float32)
            row_tot = jnp.sum(mask_f32, axis=1, keepdims=True)
            row_off = jax.lax.dot_general(
                lo32, row_tot, (((1,), (0,)), ((), ())),
                preferred_element_type=jnp.float32)
            return within + row_off

        eq_pre = excl_prefix(eq.astype(jnp.float32))
        sel_c = jnp.logical_or(gt, jnp.logical_and(eq, eq_pre < r))
        sel = jnp.where(compress == 1, sel_c.astype(jnp.int32),
                        (idx2d < _BUDGET).astype(jnp.int32))
        pos_f = excl_prefix(sel.astype(jnp.float32))
        pos = jnp.where(compress == 1, pos_f.astype(jnp.int32), idx2d)

        sel_ref[...] = sel
        pos_ref[...] = pos
        out3 = jnp.where(compress == 1, jnp.int32(_BUDGET), actual_len)
        scal_ref[...] = jnp.concatenate(
            [jnp.full((1, 128), compress, jnp.int32),
             jnp.full((1, 128), out3, jnp.int32),
             jnp.full((1, 128), actual_len, jnp.int32),
             jnp.zeros((5, 128), jnp.int32)], axis=0)


def _tc_select(k3d, hash_arr):
    return pl.pallas_call(
        _tc_select_body,
        grid=(_GRID,),
        in_specs=[
            pl.BlockSpec(memory_space=pltpu.SMEM),
            pl.BlockSpec((2, 128, _D), lambda g: (g, 0, 0)),
        ],
        out_specs=[
            pl.BlockSpec((32, 128), lambda g: (0, 0)),
            pl.BlockSpec((32, 128), lambda g: (0, 0)),
            pl.BlockSpec((8, 128), lambda g: (0, 0)),
        ],
        out_shape=[
            jax.ShapeDtypeStruct((32, 128), jnp.int32),
            jax.ShapeDtypeStruct((32, 128), jnp.int32),
            jax.ShapeDtypeStruct((8, 128), jnp.int32),
        ],
        scratch_shapes=[pltpu.VMEM((_GRID, 2, 128), jnp.int32)],
    )(hash_arr, k3d)


_NW = 32
_HEAD_PW = _BUDGET // _NW
_TAIL_PW = (_S - _BUDGET) // _NW
_CH = 8


def _sc_gather_body(k_hbm, v_hbm, sel_hbm, pos_hbm, scal_hbm,
                    outk_hbm, outv_hbm,
                    sel_v, pos_v, map_v, scal_v, bufk, bufv,
                    sem1, sem2):
    wid = lax.axis_index("s") * 2 + lax.axis_index("c")

    pltpu.sync_copy(sel_hbm, sel_v)
    pltpu.sync_copy(pos_hbm, pos_v)
    pltpu.sync_copy(scal_hbm, scal_v)

    lane = lax.iota(jnp.int32, 16)
    flag = scal_v[...][0]

    def compact(b, _):
        s16 = sel_v[pl.ds(b * 16, 16)]
        p16 = pos_v[pl.ds(b * 16, 16)]
        g16 = lane + b * 16
        plsc.store_scatter(map_v, [p16], g16, mask=s16 > 0)
        return 0

    lax.fori_loop(0, _S // 16, compact, 0, unroll=8)

    base = wid * _HEAD_PW
    for c in range(_HEAD_PW // _CH):
        off = base + c * _CH
        idx = map_v.at[pl.ds(off, _CH)]
        cp1 = pltpu.async_copy(k_hbm.at[idx], bufk, sem1)
        cp2 = pltpu.async_copy(v_hbm.at[idx], bufv, sem2)
        cp1.wait()
        pltpu.sync_copy(bufk, outk_hbm.at[pl.ds(off, _CH)])
        cp2.wait()
        pltpu.sync_copy(bufv, outv_hbm.at[pl.ds(off, _CH)])

    tbase = _BUDGET + wid * _TAIL_PW

    @pl.when(flag > 0)
    def _zero_tail():
        zeros16 = jnp.zeros((16,), jnp.float32)

        def zrow(j, _):
            for i in range(_CH):
                bufk[i, pl.ds(j * 16, 16)] = zeros16
            return 0

        lax.fori_loop(0, _D // 16, zrow, 0, unroll=8)
        for t in range(_TAIL_PW // _CH):
            off = tbase + t * _CH
            pltpu.sync_copy(bufk, outk_hbm.at[pl.ds(off, _CH)])
            pltpu.sync_copy(bufk, outv_hbm.at[pl.ds(off, _CH)])

    @pl.when(flag == 0)
    def _copy_tail():
        for t in range(_TAIL_PW // _CH):
            off = tbase + t * _CH
            pltpu.sync_copy(k_hbm.at[pl.ds(off, _CH)], bufk)
            pltpu.sync_copy(bufk, outk_hbm.at[pl.ds(off, _CH)])
            pltpu.sync_copy(v_hbm.at[pl.ds(off, _CH)], bufv)
            pltpu.sync_copy(bufv, outv_hbm.at[pl.ds(off, _CH)])


def _sc_gather(k2, v2, sel, pos, scal16):
    mesh = plsc.VectorSubcoreMesh(core_axis_name="c", subcore_axis_name="s")
    fn = functools.partial(
        pl.kernel,
        out_type=[jax.ShapeDtypeStruct((_S, _D), jnp.float32),
                  jax.ShapeDtypeStruct((_S, _D), jnp.float32)],
        mesh=mesh,
        scratch_types=[
            pltpu.VMEM((_S,), jnp.int32),
            pltpu.VMEM((_S,), jnp.int32),
            pltpu.VMEM((_BUDGET,), jnp.int32),
            pltpu.VMEM((16,), jnp.int32),
            pltpu.VMEM((_CH, _D), jnp.float32),
            pltpu.VMEM((_CH, _D), jnp.float32),
            pltpu.SemaphoreType.DMA,
            pltpu.SemaphoreType.DMA,
        ],
        compiler_params=pltpu.CompilerParams(needs_layout_passes=False),
    )(_sc_gather_body)
    return fn(k2, v2, sel, pos, scal16)


def kernel(q, k, v):
    del q
    k2 = k.reshape(_S, _D)
    v2 = v.reshape(_S, _D)
    data_hash = jnp.sum(k).astype(jnp.int32)
    sel, pos, scal = _tc_select(k.reshape(32, 128, _D),
                                data_hash.reshape(1))
    scal16 = scal.reshape(-1)[:16]
    outk, outv = _sc_gather(k2, v2, sel.reshape(_S), pos.reshape(_S), scal16)
    return (outk.reshape(_S, 32, 128), outv.reshape(_S, 32, 128),
            scal[1, 0], scal[2, 0])


# speedup vs baseline: 3.4121x; 3.4121x over previous
import functools

import jax
import jax.numpy as jnp
import numpy as np
from jax import lax
from jax.experimental import pallas as pl
from jax.experimental.pallas import tpu as pltpu
from jax.experimental.pallas import tpu_sc as plsc

_BUDGET = 1024
_S = 4096
_D = 4096
_GRID = 8


def _tf2x32(k0, k1, x0, x1):
    c = np.uint32(0x1BD11BDA)
    ks2 = k0 ^ k1 ^ c
    x0 = x0 + k0
    x1 = x1 + k1
    rotations = ((13, 15, 26, 6), (17, 29, 16, 24))
    keys = ((k1, ks2), (ks2, k0), (k0, k1), (k1, ks2), (ks2, k0))
    for i in range(5):
        for r in rotations[i % 2]:
            x0 = x0 + x1
            x1 = (x1 << np.uint32(r)) | (x1 >> np.uint32(32 - r))
            x1 = x1 ^ x0
        ka, kb = keys[i]
        x0 = x0 + ka
        x1 = x1 + kb + np.uint32(i + 1)
    return x0, x1


def _tc_select_body(hash_ref, k_ref, sel_ref, pos_ref, scal_ref, valid_scr):
    g = pl.program_id(0)
    blk = k_ref[...]
    val = jnp.any(blk != 0.0, axis=(1, 2))
    valid_scr[g] = val.reshape(4, 128).astype(jnp.int32)

    @pl.when(g == pl.num_programs(0) - 1)
    def _selection():
        valid = valid_scr[...].reshape(32, 128)
        actual_len = jnp.sum(valid)
        compress = (actual_len > _BUDGET).astype(jnp.int32)

        h_u = hash_ref[0].astype(jnp.uint32)
        fk0, fk1 = _tf2x32(jnp.uint32(0), jnp.uint32(0), jnp.uint32(0), h_u)
        idx2d = (lax.broadcasted_iota(jnp.int32, (32, 128), 0) * 128
                 + lax.broadcasted_iota(jnp.int32, (32, 128), 1))
        b1, b2 = _tf2x32(fk0, fk1,
                         jnp.zeros((32, 128), jnp.uint32),
                         idx2d.astype(jnp.uint32))
        bits = b1 ^ b2
        fbits = (bits >> np.uint32(9)) | np.uint32(0x3F800000)
        noise = lax.bitcast_convert_type(fbits, jnp.float32) - 1.0
        noise = jnp.where(valid == 0, -jnp.inf, noise)
        noise = jnp.where(idx2d == 0, jnp.inf, noise)

        def bs_step(_, lohi):
            lo, hi = lohi
            mid = (lo + hi) // 2
            t = lax.bitcast_convert_type(
                jnp.full((32, 128), mid, jnp.int32), jnp.float32)
            cnt = jnp.sum((noise > t).astype(jnp.int32))
            pred = cnt < _BUDGET
            return (jnp.where(pred, lo, mid + 1), jnp.where(pred, mid, hi))

        lo, _hi = lax.fori_loop(0, 30, bs_step,
                                (jnp.int32(0), jnp.int32(0x3F800000)))
        tvec = lax.bitcast_convert_type(
            jnp.full((32, 128), lo, jnp.int32), jnp.float32)
        gt = noise > tvec
        eq = noise == tvec
        n_gt = jnp.sum(gt.astype(jnp.int32))
        r = (_BUDGET - n_gt).astype(jnp.float32)

        up128 = (lax.broadcasted_iota(jnp.int32, (128, 128), 0)
                 < lax.broadcasted_iota(jnp.int32, (128, 128), 1)
                 ).astype(jnp.float32)
        lo32 = (lax.broadcasted_iota(jnp.int32, (32, 32), 1)
                < lax.broadcasted_iota(jnp.int32, (32, 32), 0)
                ).astype(jnp.float32)

        def excl_prefix(mask_f32):
            within = jax.lax.dot_general(
                mask_f32, up128, (((1,), (0,)), ((), ())),
                preferred_element_type=jnp.float32)
            row_tot = jnp.sum(mask_f32, axis=1, keepdims=True)
            row_off = jax.lax.dot_general(
                lo32, row_tot, (((1,), (0,)), ((), ())),
                preferred_element_type=jnp.float32)
            return within + row_off

        eq_pre = excl_prefix(eq.astype(jnp.float32))
        sel_c = jnp.logical_or(gt, jnp.logical_and(eq, eq_pre < r))
        sel = jnp.where(compress == 1, sel_c.astype(jnp.int32),
                        (idx2d < _BUDGET).astype(jnp.int32))
        pos_f = excl_prefix(sel.astype(jnp.float32))
        pos = jnp.where(compress == 1, pos_f.astype(jnp.int32), idx2d)

        sel_ref[...] = sel
        pos_ref[...] = pos
        out3 = jnp.where(compress == 1, jnp.int32(_BUDGET), actual_len)
        scal_ref[...] = jnp.concatenate(
            [jnp.full((1, 128), compress, jnp.int32),
             jnp.full((1, 128), out3, jnp.int32),
             jnp.full((1, 128), actual_len, jnp.int32),
             jnp.zeros((5, 128), jnp.int32)], axis=0)


def _tc_select(k3d, hash_arr):
    return pl.pallas_call(
        _tc_select_body,
        grid=(_GRID,),
        in_specs=[
            pl.BlockSpec(memory_space=pltpu.SMEM),
            pl.BlockSpec((512, 32, 128), lambda g: (g, 0, 0)),
        ],
        out_specs=[
            pl.BlockSpec((32, 128), lambda g: (0, 0)),
            pl.BlockSpec((32, 128), lambda g: (0, 0)),
            pl.BlockSpec((8, 128), lambda g: (0, 0)),
        ],
        out_shape=[
            jax.ShapeDtypeStruct((32, 128), jnp.int32),
            jax.ShapeDtypeStruct((32, 128), jnp.int32),
            jax.ShapeDtypeStruct((8, 128), jnp.int32),
        ],
        scratch_shapes=[pltpu.VMEM((_GRID, 4, 128), jnp.int32)],
    )(hash_arr, k3d)


_NW = 32
_HEAD_PW = _BUDGET // _NW
_TAIL_PW = (_S - _BUDGET) // _NW
_CH = 4
_NCH = _HEAD_PW // _CH
_ZCH = 4


def _sc_gather_body(k_hbm, v_hbm, sel_hbm, pos_hbm, scal_hbm,
                    outk_hbm, outv_hbm,
                    sel_v, pos_v, map_v, scal_v,
                    bufk0, bufk1, bufv0, bufv1, zbuf,
                    gsem, wsem, zsem):
    wid = lax.axis_index("s") * 2 + lax.axis_index("c")
    tbase = _BUDGET + wid * _TAIL_PW

    pltpu.sync_copy(scal_hbm, scal_v)
    flag = scal_v[...][0]

    @pl.when(flag > 0)
    def _fire_zeros():
        zeros16 = jnp.zeros((16,), jnp.float32)

        def zrow(j, _):
            for i in range(_ZCH):
                for t in range(128 // 16):
                    zbuf[i, j, pl.ds(t * 16, 16)] = zeros16
            return 0

        lax.fori_loop(0, 32, zrow, 0)
        for t in range(_TAIL_PW // _ZCH):
            off = tbase + t * _ZCH
            pltpu.make_async_copy(zbuf, outk_hbm.at[pl.ds(off, _ZCH)], zsem).start()
            pltpu.make_async_copy(zbuf, outv_hbm.at[pl.ds(off, _ZCH)], zsem).start()

    pltpu.sync_copy(sel_hbm, sel_v)
    pltpu.sync_copy(pos_hbm, pos_v)

    lane = lax.iota(jnp.int32, 16)

    def compact(b, _):
        s16 = sel_v[pl.ds(b * 16, 16)]
        p16 = pos_v[pl.ds(b * 16, 16)]
        g16 = lane + b * 16
        plsc.store_scatter(map_v, [p16 // _CH, p16 % _CH], g16, mask=s16 > 0)
        return 0

    lax.fori_loop(0, _S // 16, compact, 0, unroll=8)

    base = wid * _HEAD_PW
    crow0 = base // _CH
    bufks = (bufk0, bufk1)
    bufvs = (bufv0, bufv1)

    def start_gather(c):
        idx = map_v.at[crow0 + c]
        gk = pltpu.make_async_copy(k_hbm.at[idx], bufks[c % 2], gsem)
        gv = pltpu.make_async_copy(v_hbm.at[idx], bufvs[c % 2], gsem)
        gk.start()
        gv.start()
        return gk, gv

    g_cur = start_gather(0)
    w_prev = None
    for c in range(_NCH):
        if c + 1 < _NCH:
            if w_prev is not None:
                w_prev[0].wait()
                w_prev[1].wait()
            g_next = start_gather(c + 1)
        else:
            g_next = None
        g_cur[0].wait()
        g_cur[1].wait()
        off = base + c * _CH
        wk = pltpu.make_async_copy(bufks[c % 2], outk_hbm.at[pl.ds(off, _CH)], wsem)
        wv = pltpu.make_async_copy(bufvs[c % 2], outv_hbm.at[pl.ds(off, _CH)], wsem)
        wk.start()
        wv.start()
        w_prev = (wk, wv)
        g_cur = g_next
    w_prev[0].wait()
    w_prev[1].wait()

    @pl.when(flag > 0)
    def _drain_zeros():
        for t in range(_TAIL_PW // _ZCH):
            off = tbase + t * _ZCH
            pltpu.make_async_copy(zbuf, outk_hbm.at[pl.ds(off, _ZCH)], zsem).wait()
            pltpu.make_async_copy(zbuf, outv_hbm.at[pl.ds(off, _ZCH)], zsem).wait()

    @pl.when(flag == 0)
    def _copy_tail():
        for t in range(_TAIL_PW // _CH):
            off = tbase + t * _CH
            pltpu.sync_copy(k_hbm.at[pl.ds(off, _CH)], bufk0)
            pltpu.sync_copy(bufk0, outk_hbm.at[pl.ds(off, _CH)])
            pltpu.sync_copy(v_hbm.at[pl.ds(off, _CH)], bufv0)
            pltpu.sync_copy(bufv0, outv_hbm.at[pl.ds(off, _CH)])


def _sc_gather(k2, v2, sel, pos, scal16):
    mesh = plsc.VectorSubcoreMesh(core_axis_name="c", subcore_axis_name="s")
    fn = functools.partial(
        pl.kernel,
        out_type=[jax.ShapeDtypeStruct((_S, 32, 128), jnp.float32),
                  jax.ShapeDtypeStruct((_S, 32, 128), jnp.float32)],
        mesh=mesh,
        scratch_types=[
            pltpu.VMEM((_S,), jnp.int32),
            pltpu.VMEM((_S,), jnp.int32),
            pltpu.VMEM((_BUDGET // _CH, _CH), jnp.int32),
            pltpu.VMEM((16,), jnp.int32),
            pltpu.VMEM((_CH, 32, 128), jnp.float32),
            pltpu.VMEM((_CH, 32, 128), jnp.float32),
            pltpu.VMEM((_CH, 32, 128), jnp.float32),
            pltpu.VMEM((_CH, 32, 128), jnp.float32),
            pltpu.VMEM((_ZCH, 32, 128), jnp.float32),
            pltpu.SemaphoreType.DMA,
            pltpu.SemaphoreType.DMA,
            pltpu.SemaphoreType.DMA,
        ],
        compiler_params=pltpu.CompilerParams(needs_layout_passes=False),
    )(_sc_gather_body)
    return fn(k2, v2, sel, pos, scal16)


def kernel(q, k, v):
    del q
    data_hash = jnp.sum(k).astype(jnp.int32)
    sel, pos, scal = _tc_select(k, data_hash.reshape(1))
    scal16 = scal.reshape(-1)[:16]
    outk, outv = _sc_gather(k, v, sel.reshape(_S), pos.reshape(_S), scal16)
    return (outk, outv, scal[1, 0], scal[2, 0])
